# baseline (speedup 1.0000x reference)
import functools

import jax
import jax.numpy as jnp
from jax import lax
from jax.experimental import pallas as pl
from jax.experimental.pallas import tpu as pltpu
from jax.experimental.pallas import tpu_sc as plsc

N = 10000
E = 320000
D = 128
K = 5000

NW = 32
B = 128
CH = 80
IB = 40
NSTAGE = CH // IB
EP = NW * CH * B
T8 = 10240
K8 = 5120
TR_N = N
TR_K = K

@functools.lru_cache(maxsize=None)
def _make_edge_sum(Dd, T):
    ZR = T // 16
    mesh = plsc.VectorSubcoreMesh(core_axis_name="c", subcore_axis_name="s")

    @functools.partial(
        pl.kernel,
        out_type=jax.ShapeDtypeStruct((2, T, Dd), jnp.float32),
        mesh=mesh,
        name=f"edge_sum_{Dd}_{T}",
        scratch_types=[
            pltpu.VMEM((IB + 2, B), jnp.int32),
            pltpu.VMEM((IB, B), jnp.int32),
            pltpu.VMEM((2, B, Dd), jnp.float32),
            pltpu.VMEM_SHARED((T, Dd), jnp.float32),
            pltpu.SemaphoreType.DMA,
            pltpu.SemaphoreType.DMA,
        ],
    )
    def k(table, gidx, sidx, zrows, out, gbuf, sbuf, rows, acc, gsem, ssem):
        c = lax.axis_index("c")
        s = lax.axis_index("s")
        wid = s * 2 + c
        pltpu.sync_copy(zrows.at[pl.ds(s * ZR, ZR)], acc.at[pl.ds(s * ZR, ZR)])
        plsc.subcore_barrier()

        def gather(g, b):
            pltpu.async_copy(table.at[gbuf.at[g]], rows.at[b], gsem)

        def gwait():
            pltpu.make_async_copy(table.at[gbuf.at[0]], rows.at[0], gsem).wait()

        def scat(g, b):
            pltpu.async_copy(rows.at[b], acc.at[sbuf.at[g]], ssem, add=True)

        def swait():
            pltpu.make_async_copy(rows.at[0], acc.at[sbuf.at[0]], ssem).wait()

        def body(h, carry):
            g = h * 2
            gwait()
            gather(g + 1, 1)
            pltpu.sync_copy(rows.at[0], acc.at[sbuf.at[g]], add=True)
            gwait()
            gather(g + 2, 0)
            pltpu.sync_copy(rows.at[1], acc.at[sbuf.at[g + 1]], add=True)
            return carry

        for t in range(NSTAGE):
            pltpu.sync_copy(gidx.at[wid, t], gbuf)
            pltpu.sync_copy(sidx.at[wid, t], sbuf)
            gather(0, 0)
            lax.fori_loop(0, IB // 2, body, 0)
            gwait()
        plsc.subcore_barrier()
        pltpu.sync_copy(acc.at[pl.ds(s * ZR, ZR)], out.at[c, pl.ds(s * ZR, ZR)])

    return k


def _edge128_N(*a):
    return _make_edge_sum(D, T8)(*a)


def _edge128_K(*a):
    return _make_edge_sum(D, K8)(*a)


@functools.lru_cache(maxsize=None)
def _make_count_hist(T):
    ZR = T // 16
    mesh = plsc.VectorSubcoreMesh(core_axis_name="c", subcore_axis_name="s")

    @functools.partial(
        pl.kernel,
        out_type=jax.ShapeDtypeStruct((2, T, 16), jnp.float32),
        mesh=mesh,
        name=f"count_hist_{T}",
        scratch_types=[
            pltpu.VMEM((IB, B), jnp.int32),
            pltpu.VMEM((B, 16), jnp.float32),
            pltpu.VMEM_SHARED((T, 16), jnp.float32),
        ],
    )
    def k(sidx, zrows, out, sbuf, rows, acc):
        c = lax.axis_index("c")
        s = lax.axis_index("s")
        wid = s * 2 + c
        e0 = jnp.maximum(1 - lax.iota(jnp.int32, 16), 0).astype(jnp.float32)

        def fill(i, carry):
            rows[i, :] = e0
            return carry

        lax.fori_loop(0, B, fill, 0)
        pltpu.sync_copy(zrows.at[pl.ds(s * ZR, ZR)], acc.at[pl.ds(s * ZR, ZR)])
        plsc.subcore_barrier()

        def body(g, carry):
            pltpu.sync_copy(rows, acc.at[sbuf.at[g]], add=True)
            return carry

        for t in range(NSTAGE):
            pltpu.sync_copy(sidx.at[wid, t], sbuf)
            lax.fori_loop(0, IB, body, 0)
        plsc.subcore_barrier()
        pltpu.sync_copy(acc.at[pl.ds(s * ZR, ZR)], out.at[c, pl.ds(s * ZR, ZR)])

    return k


_R = 640


def _dot(a, b):
    return jnp.dot(a, b, precision=lax.Precision.HIGHEST,
                   preferred_element_type=jnp.float32)


def _sage_body(p_ref, c_ref, x_ref, wl_ref, wr_ref, b_ref, o_ref):
    ssum = p_ref[0] + p_ref[1]
    cnt = c_ref[0, :, 0] + c_ref[1, :, 0]
    mean = ssum / jnp.maximum(cnt, 1.0)[:, None]
    o_ref[...] = _dot(mean, wl_ref[...]) + _dot(x_ref[...], wr_ref[...]) + b_ref[0:1, :]


def _sage_res_body(p_ref, c_ref, x_ref, wl_ref, wr_ref, b_ref, r_ref, o_ref):
    ssum = p_ref[0] + p_ref[1]
    cnt = c_ref[0, :, 0] + c_ref[1, :, 0]
    mean = ssum / jnp.maximum(cnt, 1.0)[:, None]
    o_ref[...] = (_dot(mean, wl_ref[...]) + _dot(x_ref[...], wr_ref[...])
                  + b_ref[0:1, :] + r_ref[...])


def _tc_sage(T, residual=False):
    grid = (T // _R,)
    in_specs = [
        pl.BlockSpec((2, _R, D), lambda i: (0, i, 0)),
        pl.BlockSpec((2, _R, 16), lambda i: (0, i, 0)),
        pl.BlockSpec((_R, D), lambda i: (i, 0)),
        pl.BlockSpec((D, D), lambda i: (0, 0)),
        pl.BlockSpec((D, D), lambda i: (0, 0)),
        pl.BlockSpec((8, D), lambda i: (0, 0)),
    ]
    body = _sage_body
    if residual:
        in_specs.append(pl.BlockSpec((_R, D), lambda i: (i, 0)))
        body = _sage_res_body
    return pl.pallas_call(
        body,
        grid=grid,
        in_specs=in_specs,
        out_specs=pl.BlockSpec((_R, D), lambda i: (i, 0)),
        out_shape=jax.ShapeDtypeStruct((T, D), jnp.float32),
    )


_sage_N = _tc_sage(T8)
_sage_K = _tc_sage(K8)
_sage_N_res = _tc_sage(T8, residual=True)


def _wconv_score_body(p_ref, b_ref, pn_ref, hw_ref, sc_ref):
    hw = b_ref[:, 0:1] * (p_ref[0] + p_ref[1])
    hw_ref[...] = hw
    sc_ref[...] = jnp.tanh(_dot(hw, pn_ref[...]))


_wconv_score = pl.pallas_call(
    _wconv_score_body,
    grid=(T8 // _R,),
    in_specs=[
        pl.BlockSpec((2, _R, D), lambda i: (0, i, 0)),
        pl.BlockSpec((_R, 16), lambda i: (i, 0)),
        pl.BlockSpec((D, 16), lambda i: (0, 0)),
    ],
    out_specs=[
        pl.BlockSpec((_R, D), lambda i: (i, 0)),
        pl.BlockSpec((_R, 16), lambda i: (i, 0)),
    ],
    out_shape=[
        jax.ShapeDtypeStruct((T8, D), jnp.float32),
        jax.ShapeDtypeStruct((T8, 16), jnp.float32),
    ],
)


def _acomb_body(p_ref, a_ref, o_ref):
    o_ref[...] = a_ref[:, 0:1] * (p_ref[0] + p_ref[1])


_acomb = pl.pallas_call(
    _acomb_body,
    grid=(T8 // _R,),
    in_specs=[
        pl.BlockSpec((2, _R, D), lambda i: (0, i, 0)),
        pl.BlockSpec((_R, 16), lambda i: (i, 0)),
    ],
    out_specs=pl.BlockSpec((_R, D), lambda i: (i, 0)),
    out_shape=jax.ShapeDtypeStruct((T8, D), jnp.float32),
)


def _col16(v, T):
    out = jnp.zeros((T, 16), jnp.float32)
    return out.at[:, 0].set(v)


def kernel(x, edge_index, batch, weights,
           Wl_d00, bl_d00, Wr_d00, Wl_d01, bl_d01, Wr_d01,
           Wl_bt0, bl_bt0, Wr_bt0, Wl_bt1, bl_bt1, Wr_bt1,
           Wl_u00, bl_u00, Wr_u00, Wl_u01, bl_u01, Wr_u01, pool_p):
    f32 = jnp.float32
    src = edge_index[0].astype(jnp.int32)
    dst = edge_index[1].astype(jnp.int32)
    pad = N + (jnp.arange(EP - E, dtype=jnp.int32) % (T8 - N))
    srcp = jnp.concatenate([src, pad])
    dstp = jnp.concatenate([dst, pad])

    def g3(idx):
        r = idx.reshape(NW, NSTAGE, IB, B)
        return jnp.concatenate(
            [r, jnp.zeros((NW, NSTAGE, 2, B), jnp.int32)], axis=2)

    def s3(idx):
        return idx.reshape(NW, NSTAGE, IB, B)

    src_g, src_s = g3(srcp), s3(srcp)
    dst_g, dst_s = g3(dstp), s3(dstp)

    x_pad = jnp.zeros((T8, D), f32).at[:N].set(x)
    _Z128_N = jnp.zeros((T8, D), f32)
    _Z16_N = jnp.zeros((T8, 16), f32)

    def bias8(b):
        return jnp.broadcast_to(b[None, :], (8, D))

    Cdst = _make_count_hist(T8)(dst_s, _Z16_N)
    Cdeg = _make_count_hist(T8)(src_s, _Z16_N)

    P = _edge128_N(x_pad, src_g, dst_s, _Z128_N)
    h1 = _sage_N(P, Cdst, x_pad, Wl_d00.T, Wr_d00.T, bias8(bl_d00))
    P = _edge128_N(h1, src_g, dst_s, _Z128_N)
    h2 = _sage_N(P, Cdst, h1, Wl_d01.T, Wr_d01.T, bias8(bl_d01))

    deg = Cdeg[0, :, 0] + Cdeg[1, :, 0]
    w0p = jnp.zeros((T8,), f32).at[:N].set(weights[:, 0])
    a = w0p / deg
    a_tab = jnp.zeros((T8, D), f32).at[:, 0].set(a)
    Ca = _edge128_N(a_tab, src_g, dst_s, _Z128_N)
    aggr = Ca[0, :, 0] + Ca[1, :, 0] + 1e-12
    b = 1.0 / aggr

    pn = pool_p / jnp.linalg.norm(pool_p)
    pn16 = jnp.zeros((D, 16), f32).at[:, 0].set(pn)
    Pw = _edge128_N(a[:, None] * h2, src_g, dst_s, _Z128_N)
    hw, score16 = _wconv_score(Pw, _col16(b, T8), pn16)
    score = score16[:N, 0]

    topv, _perm = lax.top_k(score, K)
    tau = topv[K - 1]
    gt = score > tau
    n_gt = jnp.sum(gt.astype(jnp.int32))
    tie = score == tau
    tie_rank = jnp.cumsum(tie.astype(jnp.int32))
    keep = gt | (tie & (tie_rank <= (K - n_gt)))
    keepf = jnp.zeros((T8,), f32).at[:N].set(keep.astype(f32))

    hp_pad = hw * (score16[:, 0] * keepf)[:, None]

    keep_tab = jnp.zeros((T8, D), f32).at[:, 0].set(keepf)
    C2p = _edge128_N(keep_tab, src_g, dst_s, _Z128_N)
    C2 = C2p[:, :, :16]

    P = _edge128_N(hp_pad, src_g, dst_s, _Z128_N)
    hb = _sage_N(P, C2, hp_pad, Wl_bt0.T, Wr_bt0.T, bias8(bl_bt0))
    hb = hb * keepf[:, None]
    P = _edge128_N(hb, src_g, dst_s, _Z128_N)
    hb = _sage_N(P, C2, hb, Wl_bt1.T, Wr_bt1.T, bias8(bl_bt1))

    hu_nodes = hb * keepf[:, None]
    Pr = _edge128_N(b[:, None] * hu_nodes, dst_g, src_s, _Z128_N)
    hu = _acomb(Pr, _col16(a, T8))

    P = _edge128_N(hu, src_g, dst_s, _Z128_N)
    h = _sage_N(P, Cdst, hu, Wl_u00.T, Wr_u00.T, bias8(bl_u00))
    P = _edge128_N(h, src_g, dst_s, _Z128_N)
    out = _sage_N_res(P, Cdst, h, Wl_u01.T, Wr_u01.T, bias8(bl_u01), h2)
    return out[:N]


# speedup vs baseline: 2.2288x; 2.2288x over previous
import functools

import jax
import jax.numpy as jnp
from jax import lax
from jax.experimental import pallas as pl
from jax.experimental.pallas import tpu as pltpu
from jax.experimental.pallas import tpu_sc as plsc

N = 10000
E = 320000
D = 128
K = 5000

NW = 32
B = 128
CH = 80
IB = 40
NSTAGE = CH // IB
EP = NW * CH * B
T8 = 10240
K8 = 5120
TR_N = N
TR_K = K

@functools.lru_cache(maxsize=None)
def _make_edge_sum(Dd, T):
    ZR = T // 16
    mesh = plsc.VectorSubcoreMesh(core_axis_name="c", subcore_axis_name="s")

    @functools.partial(
        pl.kernel,
        out_type=jax.ShapeDtypeStruct((2, T, Dd), jnp.float32),
        mesh=mesh,
        name=f"edge_sum_{Dd}_{T}",
        scratch_types=[
            pltpu.VMEM((IB + 2, B), jnp.int32),
            pltpu.VMEM((IB, B), jnp.int32),
            pltpu.VMEM((2, B, Dd), jnp.float32),
            pltpu.VMEM_SHARED((T, Dd), jnp.float32),
            pltpu.SemaphoreType.DMA,
            pltpu.SemaphoreType.DMA,
        ],
    )
    def k(table, gidx, sidx, zrows, out, gbuf, sbuf, rows, acc, gsem, ssem):
        c = lax.axis_index("c")
        s = lax.axis_index("s")
        wid = s * 2 + c
        pltpu.sync_copy(zrows.at[pl.ds(s * ZR, ZR)], acc.at[pl.ds(s * ZR, ZR)])
        plsc.subcore_barrier()

        def gather(g, b):
            pltpu.async_copy(table.at[gbuf.at[g]], rows.at[b], gsem)

        def gwait():
            pltpu.make_async_copy(table.at[gbuf.at[0]], rows.at[0], gsem).wait()

        def scat(g, b):
            pltpu.async_copy(rows.at[b], acc.at[sbuf.at[g]], ssem, add=True)

        def swait():
            pltpu.make_async_copy(rows.at[0], acc.at[sbuf.at[0]], ssem).wait()

        def body(g, carry):
            pltpu.async_copy(table.at[gbuf.at[g]], rows.at[0], gsem).wait()
            pltpu.sync_copy(rows.at[0], acc.at[sbuf.at[g]], add=True)
            return carry

        for t in range(NSTAGE):
            pltpu.sync_copy(gidx.at[wid, t], gbuf)
            pltpu.sync_copy(sidx.at[wid, t], sbuf)
            lax.fori_loop(0, IB, body, 0)
        plsc.subcore_barrier()
        pltpu.sync_copy(acc.at[pl.ds(s * ZR, ZR)], out.at[c, pl.ds(s * ZR, ZR)])

    return k


def _edge128_N(*a):
    return _make_edge_sum(D, T8)(*a)


def _edge128_K(*a):
    return _make_edge_sum(D, K8)(*a)


@functools.lru_cache(maxsize=None)
def _make_count_hist(T):
    ZR = T // 16
    mesh = plsc.VectorSubcoreMesh(core_axis_name="c", subcore_axis_name="s")

    @functools.partial(
        pl.kernel,
        out_type=jax.ShapeDtypeStruct((2, T, 16), jnp.float32),
        mesh=mesh,
        name=f"count_hist_{T}",
        scratch_types=[
            pltpu.VMEM((IB, B), jnp.int32),
            pltpu.VMEM((B, 16), jnp.float32),
            pltpu.VMEM_SHARED((T, 16), jnp.float32),
        ],
    )
    def k(sidx, zrows, out, sbuf, rows, acc):
        c = lax.axis_index("c")
        s = lax.axis_index("s")
        wid = s * 2 + c
        e0 = jnp.maximum(1 - lax.iota(jnp.int32, 16), 0).astype(jnp.float32)

        def fill(i, carry):
            rows[i, :] = e0
            return carry

        lax.fori_loop(0, B, fill, 0)
        pltpu.sync_copy(zrows.at[pl.ds(s * ZR, ZR)], acc.at[pl.ds(s * ZR, ZR)])
        plsc.subcore_barrier()

        def body(g, carry):
            pltpu.sync_copy(rows, acc.at[sbuf.at[g]], add=True)
            return carry

        for t in range(NSTAGE):
            pltpu.sync_copy(sidx.at[wid, t], sbuf)
            lax.fori_loop(0, IB, body, 0)
        plsc.subcore_barrier()
        pltpu.sync_copy(acc.at[pl.ds(s * ZR, ZR)], out.at[c, pl.ds(s * ZR, ZR)])

    return k


_R = 640


def _dot(a, b):
    return jnp.dot(a, b, precision=lax.Precision.HIGHEST,
                   preferred_element_type=jnp.float32)


def _sage_body(p_ref, c_ref, x_ref, wl_ref, wr_ref, b_ref, o_ref):
    ssum = p_ref[0] + p_ref[1]
    cnt = c_ref[0, :, 0] + c_ref[1, :, 0]
    mean = ssum / jnp.maximum(cnt, 1.0)[:, None]
    o_ref[...] = _dot(mean, wl_ref[...]) + _dot(x_ref[...], wr_ref[...]) + b_ref[0:1, :]


def _sage_res_body(p_ref, c_ref, x_ref, wl_ref, wr_ref, b_ref, r_ref, o_ref):
    ssum = p_ref[0] + p_ref[1]
    cnt = c_ref[0, :, 0] + c_ref[1, :, 0]
    mean = ssum / jnp.maximum(cnt, 1.0)[:, None]
    o_ref[...] = (_dot(mean, wl_ref[...]) + _dot(x_ref[...], wr_ref[...])
                  + b_ref[0:1, :] + r_ref[...])


def _tc_sage(T, residual=False):
    grid = (T // _R,)
    in_specs = [
        pl.BlockSpec((2, _R, D), lambda i: (0, i, 0)),
        pl.BlockSpec((2, _R, 16), lambda i: (0, i, 0)),
        pl.BlockSpec((_R, D), lambda i: (i, 0)),
        pl.BlockSpec((D, D), lambda i: (0, 0)),
        pl.BlockSpec((D, D), lambda i: (0, 0)),
        pl.BlockSpec((8, D), lambda i: (0, 0)),
    ]
    body = _sage_body
    if residual:
        in_specs.append(pl.BlockSpec((_R, D), lambda i: (i, 0)))
        body = _sage_res_body
    return pl.pallas_call(
        body,
        grid=grid,
        in_specs=in_specs,
        out_specs=pl.BlockSpec((_R, D), lambda i: (i, 0)),
        out_shape=jax.ShapeDtypeStruct((T, D), jnp.float32),
    )


_sage_N = _tc_sage(T8)
_sage_K = _tc_sage(K8)
_sage_N_res = _tc_sage(T8, residual=True)


def _wconv_score_body(p_ref, b_ref, pn_ref, hw_ref, sc_ref):
    hw = b_ref[:, 0:1] * (p_ref[0] + p_ref[1])
    hw_ref[...] = hw
    sc_ref[...] = jnp.tanh(_dot(hw, pn_ref[...]))


_wconv_score = pl.pallas_call(
    _wconv_score_body,
    grid=(T8 // _R,),
    in_specs=[
        pl.BlockSpec((2, _R, D), lambda i: (0, i, 0)),
        pl.BlockSpec((_R, 16), lambda i: (i, 0)),
        pl.BlockSpec((D, 16), lambda i: (0, 0)),
    ],
    out_specs=[
        pl.BlockSpec((_R, D), lambda i: (i, 0)),
        pl.BlockSpec((_R, 16), lambda i: (i, 0)),
    ],
    out_shape=[
        jax.ShapeDtypeStruct((T8, D), jnp.float32),
        jax.ShapeDtypeStruct((T8, 16), jnp.float32),
    ],
)


def _acomb_body(p_ref, a_ref, o_ref):
    o_ref[...] = a_ref[:, 0:1] * (p_ref[0] + p_ref[1])


_acomb = pl.pallas_call(
    _acomb_body,
    grid=(T8 // _R,),
    in_specs=[
        pl.BlockSpec((2, _R, D), lambda i: (0, i, 0)),
        pl.BlockSpec((_R, 16), lambda i: (i, 0)),
    ],
    out_specs=pl.BlockSpec((_R, D), lambda i: (i, 0)),
    out_shape=jax.ShapeDtypeStruct((T8, D), jnp.float32),
)


def _col16(v, T):
    out = jnp.zeros((T, 16), jnp.float32)
    return out.at[:, 0].set(v)


def kernel(x, edge_index, batch, weights,
           Wl_d00, bl_d00, Wr_d00, Wl_d01, bl_d01, Wr_d01,
           Wl_bt0, bl_bt0, Wr_bt0, Wl_bt1, bl_bt1, Wr_bt1,
           Wl_u00, bl_u00, Wr_u00, Wl_u01, bl_u01, Wr_u01, pool_p):
    f32 = jnp.float32
    src = edge_index[0].astype(jnp.int32)
    dst = edge_index[1].astype(jnp.int32)
    pad = N + (jnp.arange(EP - E, dtype=jnp.int32) % (T8 - N))
    srcp = jnp.concatenate([src, pad])
    dstp = jnp.concatenate([dst, pad])

    def g3(idx):
        r = idx.reshape(NW, NSTAGE, IB, B)
        return jnp.concatenate(
            [r, jnp.zeros((NW, NSTAGE, 2, B), jnp.int32)], axis=2)

    def s3(idx):
        return idx.reshape(NW, NSTAGE, IB, B)

    src_g, src_s = g3(srcp), s3(srcp)
    dst_g, dst_s = g3(dstp), s3(dstp)

    x_pad = jnp.zeros((T8, D), f32).at[:N].set(x)
    _Z128_N = jnp.zeros((T8, D), f32)
    _Z16_N = jnp.zeros((T8, 16), f32)

    def bias8(b):
        return jnp.broadcast_to(b[None, :], (8, D))

    Cdst = _make_count_hist(T8)(dst_s, _Z16_N)
    Cdeg = _make_count_hist(T8)(src_s, _Z16_N)

    P = _edge128_N(x_pad, src_g, dst_s, _Z128_N)
    h1 = _sage_N(P, Cdst, x_pad, Wl_d00.T, Wr_d00.T, bias8(bl_d00))
    P = _edge128_N(h1, src_g, dst_s, _Z128_N)
    h2 = _sage_N(P, Cdst, h1, Wl_d01.T, Wr_d01.T, bias8(bl_d01))

    deg = Cdeg[0, :, 0] + Cdeg[1, :, 0]
    w0p = jnp.zeros((T8,), f32).at[:N].set(weights[:, 0])
    a = w0p / deg
    a_tab = jnp.zeros((T8, D), f32).at[:, 0].set(a)
    Ca = _edge128_N(a_tab, src_g, dst_s, _Z128_N)
    aggr = Ca[0, :, 0] + Ca[1, :, 0] + 1e-12
    b = 1.0 / aggr

    pn = pool_p / jnp.linalg.norm(pool_p)
    pn16 = jnp.zeros((D, 16), f32).at[:, 0].set(pn)
    Pw = _edge128_N(a[:, None] * h2, src_g, dst_s, _Z128_N)
    hw, score16 = _wconv_score(Pw, _col16(b, T8), pn16)
    score = score16[:N, 0]

    topv, _perm = lax.top_k(score, K)
    tau = topv[K - 1]
    gt = score > tau
    n_gt = jnp.sum(gt.astype(jnp.int32))
    tie = score == tau
    tie_rank = jnp.cumsum(tie.astype(jnp.int32))
    keep = gt | (tie & (tie_rank <= (K - n_gt)))
    keepf = jnp.zeros((T8,), f32).at[:N].set(keep.astype(f32))

    hp_pad = hw * (score16[:, 0] * keepf)[:, None]

    keep_tab = jnp.zeros((T8, D), f32).at[:, 0].set(keepf)
    C2p = _edge128_N(keep_tab, src_g, dst_s, _Z128_N)
    C2 = C2p[:, :, :16]

    P = _edge128_N(hp_pad, src_g, dst_s, _Z128_N)
    hb = _sage_N(P, C2, hp_pad, Wl_bt0.T, Wr_bt0.T, bias8(bl_bt0))
    hb = hb * keepf[:, None]
    P = _edge128_N(hb, src_g, dst_s, _Z128_N)
    hb = _sage_N(P, C2, hb, Wl_bt1.T, Wr_bt1.T, bias8(bl_bt1))

    hu_nodes = hb * keepf[:, None]
    Pr = _edge128_N(b[:, None] * hu_nodes, dst_g, src_s, _Z128_N)
    hu = _acomb(Pr, _col16(a, T8))

    P = _edge128_N(hu, src_g, dst_s, _Z128_N)
    h = _sage_N(P, Cdst, hu, Wl_u00.T, Wr_u00.T, bias8(bl_u00))
    P = _edge128_N(h, src_g, dst_s, _Z128_N)
    out = _sage_N_res(P, Cdst, h, Wl_u01.T, Wr_u01.T, bias8(bl_u01), h2)
    return out[:N]


# speedup vs baseline: 2.8345x; 1.2718x over previous
import functools

import jax
import jax.numpy as jnp
from jax import lax
from jax.experimental import pallas as pl
from jax.experimental.pallas import tpu as pltpu
from jax.experimental.pallas import tpu_sc as plsc

N = 10000
E = 320000
D = 128
K = 5000

NW = 32
B = 128
CH = 80
IB = 40
NSTAGE = CH // IB
EP = NW * CH * B
T8 = 10240
K8 = 5120
TR_N = N
TR_K = K

@functools.lru_cache(maxsize=None)
def _make_edge_sum(Dd, T):
    ZR = T // 16
    mesh = plsc.VectorSubcoreMesh(core_axis_name="c", subcore_axis_name="s")

    @functools.partial(
        pl.kernel,
        out_type=jax.ShapeDtypeStruct((2, T, Dd), jnp.float32),
        mesh=mesh,
        name=f"edge_sum_{Dd}_{T}",
        scratch_types=[
            pltpu.VMEM((IB + 2, B), jnp.int32),
            pltpu.VMEM((IB, B), jnp.int32),
            pltpu.VMEM((2, B, Dd), jnp.float32),
            pltpu.VMEM_SHARED((T, Dd), jnp.float32),
            pltpu.SemaphoreType.DMA,
            pltpu.SemaphoreType.DMA,
        ],
    )
    def k(table, gidx, sidx, zrows, out, gbuf, sbuf, rows, acc, gsem, ssem):
        c = lax.axis_index("c")
        s = lax.axis_index("s")
        wid = s * 2 + c
        pltpu.sync_copy(zrows.at[pl.ds(s * ZR, ZR)], acc.at[pl.ds(s * ZR, ZR)])
        plsc.subcore_barrier()

        def gather(g, b):
            pltpu.async_copy(table.at[gbuf.at[g]], rows.at[b], gsem)

        def gwait():
            pltpu.make_async_copy(table.at[gbuf.at[0]], rows.at[0], gsem).wait()

        def scat(g, b):
            pltpu.async_copy(rows.at[b], acc.at[sbuf.at[g]], ssem, add=True)

        def swait():
            pltpu.make_async_copy(rows.at[0], acc.at[sbuf.at[0]], ssem).wait()

        U = 20

        def body(q, carry):
            g0 = q * U
            d = pltpu.async_copy(table.at[gbuf.at[g0]], rows.at[0], gsem)
            for j in range(U):
                d.wait()
                if j + 1 < U:
                    d = pltpu.async_copy(
                        table.at[gbuf.at[g0 + j + 1]],
                        rows.at[(j + 1) % 2], gsem)
                pltpu.sync_copy(rows.at[j % 2], acc.at[sbuf.at[g0 + j]],
                                add=True)
            return carry

        for t in range(NSTAGE):
            pltpu.sync_copy(gidx.at[wid, t], gbuf)
            pltpu.sync_copy(sidx.at[wid, t], sbuf)
            lax.fori_loop(0, IB // U, body, 0)
        plsc.subcore_barrier()
        pltpu.sync_copy(acc.at[pl.ds(s * ZR, ZR)], out.at[c, pl.ds(s * ZR, ZR)])

    return k


def _edge128_N(*a):
    return _make_edge_sum(D, T8)(*a)


def _edge128_K(*a):
    return _make_edge_sum(D, K8)(*a)


@functools.lru_cache(maxsize=None)
def _make_count_hist(T):
    ZR = T // 16
    mesh = plsc.VectorSubcoreMesh(core_axis_name="c", subcore_axis_name="s")

    @functools.partial(
        pl.kernel,
        out_type=jax.ShapeDtypeStruct((2, T, 16), jnp.float32),
        mesh=mesh,
        name=f"count_hist_{T}",
        scratch_types=[
            pltpu.VMEM((IB, B), jnp.int32),
            pltpu.VMEM((B, 16), jnp.float32),
            pltpu.VMEM_SHARED((T, 16), jnp.float32),
        ],
    )
    def k(sidx, zrows, out, sbuf, rows, acc):
        c = lax.axis_index("c")
        s = lax.axis_index("s")
        wid = s * 2 + c
        e0 = jnp.maximum(1 - lax.iota(jnp.int32, 16), 0).astype(jnp.float32)

        def fill(i, carry):
            rows[i, :] = e0
            return carry

        lax.fori_loop(0, B, fill, 0)
        pltpu.sync_copy(zrows.at[pl.ds(s * ZR, ZR)], acc.at[pl.ds(s * ZR, ZR)])
        plsc.subcore_barrier()

        def body(g, carry):
            pltpu.sync_copy(rows, acc.at[sbuf.at[g]], add=True)
            return carry

        for t in range(NSTAGE):
            pltpu.sync_copy(sidx.at[wid, t], sbuf)
            lax.fori_loop(0, IB, body, 0)
        plsc.subcore_barrier()
        pltpu.sync_copy(acc.at[pl.ds(s * ZR, ZR)], out.at[c, pl.ds(s * ZR, ZR)])

    return k


_R = 640


def _dot(a, b):
    return jnp.dot(a, b, precision=lax.Precision.HIGHEST,
                   preferred_element_type=jnp.float32)


def _sage_body(p_ref, c_ref, x_ref, wl_ref, wr_ref, b_ref, o_ref):
    ssum = p_ref[0] + p_ref[1]
    cnt = c_ref[0, :, 0] + c_ref[1, :, 0]
    mean = ssum / jnp.maximum(cnt, 1.0)[:, None]
    o_ref[...] = _dot(mean, wl_ref[...]) + _dot(x_ref[...], wr_ref[...]) + b_ref[0:1, :]


def _sage_res_body(p_ref, c_ref, x_ref, wl_ref, wr_ref, b_ref, r_ref, o_ref):
    ssum = p_ref[0] + p_ref[1]
    cnt = c_ref[0, :, 0] + c_ref[1, :, 0]
    mean = ssum / jnp.maximum(cnt, 1.0)[:, None]
    o_ref[...] = (_dot(mean, wl_ref[...]) + _dot(x_ref[...], wr_ref[...])
                  + b_ref[0:1, :] + r_ref[...])


def _tc_sage(T, residual=False):
    grid = (T // _R,)
    in_specs = [
        pl.BlockSpec((2, _R, D), lambda i: (0, i, 0)),
        pl.BlockSpec((2, _R, 16), lambda i: (0, i, 0)),
        pl.BlockSpec((_R, D), lambda i: (i, 0)),
        pl.BlockSpec((D, D), lambda i: (0, 0)),
        pl.BlockSpec((D, D), lambda i: (0, 0)),
        pl.BlockSpec((8, D), lambda i: (0, 0)),
    ]
    body = _sage_body
    if residual:
        in_specs.append(pl.BlockSpec((_R, D), lambda i: (i, 0)))
        body = _sage_res_body
    return pl.pallas_call(
        body,
        grid=grid,
        in_specs=in_specs,
        out_specs=pl.BlockSpec((_R, D), lambda i: (i, 0)),
        out_shape=jax.ShapeDtypeStruct((T, D), jnp.float32),
    )


_sage_N = _tc_sage(T8)
_sage_K = _tc_sage(K8)
_sage_N_res = _tc_sage(T8, residual=True)


def _wconv_score_body(p_ref, b_ref, pn_ref, hw_ref, sc_ref):
    hw = b_ref[:, 0:1] * (p_ref[0] + p_ref[1])
    hw_ref[...] = hw
    sc_ref[...] = jnp.tanh(_dot(hw, pn_ref[...]))


_wconv_score = pl.pallas_call(
    _wconv_score_body,
    grid=(T8 // _R,),
    in_specs=[
        pl.BlockSpec((2, _R, D), lambda i: (0, i, 0)),
        pl.BlockSpec((_R, 16), lambda i: (i, 0)),
        pl.BlockSpec((D, 16), lambda i: (0, 0)),
    ],
    out_specs=[
        pl.BlockSpec((_R, D), lambda i: (i, 0)),
        pl.BlockSpec((_R, 16), lambda i: (i, 0)),
    ],
    out_shape=[
        jax.ShapeDtypeStruct((T8, D), jnp.float32),
        jax.ShapeDtypeStruct((T8, 16), jnp.float32),
    ],
)


def _acomb_body(p_ref, a_ref, o_ref):
    o_ref[...] = a_ref[:, 0:1] * (p_ref[0] + p_ref[1])


_acomb = pl.pallas_call(
    _acomb_body,
    grid=(T8 // _R,),
    in_specs=[
        pl.BlockSpec((2, _R, D), lambda i: (0, i, 0)),
        pl.BlockSpec((_R, 16), lambda i: (i, 0)),
    ],
    out_specs=pl.BlockSpec((_R, D), lambda i: (i, 0)),
    out_shape=jax.ShapeDtypeStruct((T8, D), jnp.float32),
)


def _col16(v, T):
    out = jnp.zeros((T, 16), jnp.float32)
    return out.at[:, 0].set(v)


def kernel(x, edge_index, batch, weights,
           Wl_d00, bl_d00, Wr_d00, Wl_d01, bl_d01, Wr_d01,
           Wl_bt0, bl_bt0, Wr_bt0, Wl_bt1, bl_bt1, Wr_bt1,
           Wl_u00, bl_u00, Wr_u00, Wl_u01, bl_u01, Wr_u01, pool_p):
    f32 = jnp.float32
    src = edge_index[0].astype(jnp.int32)
    dst = edge_index[1].astype(jnp.int32)
    pad = N + (jnp.arange(EP - E, dtype=jnp.int32) % (T8 - N))
    srcp = jnp.concatenate([src, pad])
    dstp = jnp.concatenate([dst, pad])

    def g3(idx):
        r = idx.reshape(NW, NSTAGE, IB, B)
        return jnp.concatenate(
            [r, jnp.zeros((NW, NSTAGE, 2, B), jnp.int32)], axis=2)

    def s3(idx):
        return idx.reshape(NW, NSTAGE, IB, B)

    src_g, src_s = g3(srcp), s3(srcp)
    dst_g, dst_s = g3(dstp), s3(dstp)

    x_pad = jnp.zeros((T8, D), f32).at[:N].set(x)
    _Z128_N = jnp.zeros((T8, D), f32)
    _Z16_N = jnp.zeros((T8, 16), f32)

    def bias8(b):
        return jnp.broadcast_to(b[None, :], (8, D))

    Cdst = _make_count_hist(T8)(dst_s, _Z16_N)
    Cdeg = _make_count_hist(T8)(src_s, _Z16_N)

    P = _edge128_N(x_pad, src_g, dst_s, _Z128_N)
    h1 = _sage_N(P, Cdst, x_pad, Wl_d00.T, Wr_d00.T, bias8(bl_d00))
    P = _edge128_N(h1, src_g, dst_s, _Z128_N)
    h2 = _sage_N(P, Cdst, h1, Wl_d01.T, Wr_d01.T, bias8(bl_d01))

    deg = Cdeg[0, :, 0] + Cdeg[1, :, 0]
    w0p = jnp.zeros((T8,), f32).at[:N].set(weights[:, 0])
    a = w0p / deg
    a_tab = jnp.zeros((T8, D), f32).at[:, 0].set(a)
    Ca = _edge128_N(a_tab, src_g, dst_s, _Z128_N)
    aggr = Ca[0, :, 0] + Ca[1, :, 0] + 1e-12
    b = 1.0 / aggr

    pn = pool_p / jnp.linalg.norm(pool_p)
    pn16 = jnp.zeros((D, 16), f32).at[:, 0].set(pn)
    Pw = _edge128_N(a[:, None] * h2, src_g, dst_s, _Z128_N)
    hw, score16 = _wconv_score(Pw, _col16(b, T8), pn16)
    score = score16[:N, 0]

    topv, _perm = lax.top_k(score, K)
    tau = topv[K - 1]
    gt = score > tau
    n_gt = jnp.sum(gt.astype(jnp.int32))
    tie = score == tau
    tie_rank = jnp.cumsum(tie.astype(jnp.int32))
    keep = gt | (tie & (tie_rank <= (K - n_gt)))
    keepf = jnp.zeros((T8,), f32).at[:N].set(keep.astype(f32))

    hp_pad = hw * (score16[:, 0] * keepf)[:, None]

    keep_tab = jnp.zeros((T8, D), f32).at[:, 0].set(keepf)
    C2p = _edge128_N(keep_tab, src_g, dst_s, _Z128_N)
    C2 = C2p[:, :, :16]

    P = _edge128_N(hp_pad, src_g, dst_s, _Z128_N)
    hb = _sage_N(P, C2, hp_pad, Wl_bt0.T, Wr_bt0.T, bias8(bl_bt0))
    hb = hb * keepf[:, None]
    P = _edge128_N(hb, src_g, dst_s, _Z128_N)
    hb = _sage_N(P, C2, hb, Wl_bt1.T, Wr_bt1.T, bias8(bl_bt1))

    hu_nodes = hb * keepf[:, None]
    Pr = _edge128_N(b[:, None] * hu_nodes, dst_g, src_s, _Z128_N)
    hu = _acomb(Pr, _col16(a, T8))

    P = _edge128_N(hu, src_g, dst_s, _Z128_N)
    h = _sage_N(P, Cdst, hu, Wl_u00.T, Wr_u00.T, bias8(bl_u00))
    P = _edge128_N(h, src_g, dst_s, _Z128_N)
    out = _sage_N_res(P, Cdst, h, Wl_u01.T, Wr_u01.T, bias8(bl_u01), h2)
    return out[:N]


# speedup vs baseline: 2.8470x; 1.0044x over previous
import functools

import jax
import jax.numpy as jnp
from jax import lax
from jax.experimental import pallas as pl
from jax.experimental.pallas import tpu as pltpu
from jax.experimental.pallas import tpu_sc as plsc

N = 10000
E = 320000
D = 128
K = 5000

NW = 32
B = 128
CH = 80
IB = 40
NSTAGE = CH // IB
EP = NW * CH * B
T8 = 10240

@functools.lru_cache(maxsize=None)
def _make_edge_sum(Dd, T):
    ZR = T // 16
    mesh = plsc.VectorSubcoreMesh(core_axis_name="c", subcore_axis_name="s")

    @functools.partial(
        pl.kernel,
        out_type=jax.ShapeDtypeStruct((2, T, Dd), jnp.float32),
        mesh=mesh,
        name=f"edge_sum_{Dd}_{T}",
        scratch_types=[
            pltpu.VMEM((IB + 2, B), jnp.int32),
            pltpu.VMEM((IB, B), jnp.int32),
            pltpu.VMEM((2, B, Dd), jnp.float32),
            pltpu.VMEM_SHARED((T, Dd), jnp.float32),
            pltpu.SemaphoreType.DMA,
        ],
    )
    def k(table, gidx, sidx, zrows, out, gbuf, sbuf, rows, acc, gsem):
        c = lax.axis_index("c")
        s = lax.axis_index("s")
        wid = s * 2 + c
        pltpu.sync_copy(zrows.at[pl.ds(s * ZR, ZR)], acc.at[pl.ds(s * ZR, ZR)])
        plsc.subcore_barrier()

        U = 20

        def body(q, carry):
            g0 = q * U
            d = pltpu.async_copy(table.at[gbuf.at[g0]], rows.at[0], gsem)
            for j in range(U):
                d.wait()
                if j + 1 < U:
                    d = pltpu.async_copy(
                        table.at[gbuf.at[g0 + j + 1]],
                        rows.at[(j + 1) % 2], gsem)
                pltpu.sync_copy(rows.at[j % 2], acc.at[sbuf.at[g0 + j]],
                                add=True)
            return carry

        for t in range(NSTAGE):
            pltpu.sync_copy(gidx.at[wid, t], gbuf)
            pltpu.sync_copy(sidx.at[wid, t], sbuf)
            lax.fori_loop(0, IB // U, body, 0)
        plsc.subcore_barrier()
        pltpu.sync_copy(acc.at[pl.ds(s * ZR, ZR)], out.at[c, pl.ds(s * ZR, ZR)])

    return k


def _edge128_N(*a):
    return _make_edge_sum(D, T8)(*a)


@functools.lru_cache(maxsize=None)
def _make_count_hist(T):
    ZR = T // 16
    mesh = plsc.VectorSubcoreMesh(core_axis_name="c", subcore_axis_name="s")

    @functools.partial(
        pl.kernel,
        out_type=jax.ShapeDtypeStruct((2, T, 16), jnp.float32),
        mesh=mesh,
        name=f"count_hist_{T}",
        scratch_types=[
            pltpu.VMEM((IB, B), jnp.int32),
            pltpu.VMEM((B, 16), jnp.float32),
            pltpu.VMEM_SHARED((T, 16), jnp.float32),
        ],
    )
    def k(sidx, zrows, out, sbuf, rows, acc):
        c = lax.axis_index("c")
        s = lax.axis_index("s")
        wid = s * 2 + c
        e0 = jnp.maximum(1 - lax.iota(jnp.int32, 16), 0).astype(jnp.float32)

        def fill(i, carry):
            rows[i, :] = e0
            return carry

        lax.fori_loop(0, B, fill, 0)
        pltpu.sync_copy(zrows.at[pl.ds(s * ZR, ZR)], acc.at[pl.ds(s * ZR, ZR)])
        plsc.subcore_barrier()

        def body(g, carry):
            pltpu.sync_copy(rows, acc.at[sbuf.at[g]], add=True)
            return carry

        for t in range(NSTAGE):
            pltpu.sync_copy(sidx.at[wid, t], sbuf)
            lax.fori_loop(0, IB, body, 0)
        plsc.subcore_barrier()
        pltpu.sync_copy(acc.at[pl.ds(s * ZR, ZR)], out.at[c, pl.ds(s * ZR, ZR)])

    return k


_R = 640


def _dot(a, b):
    return jnp.dot(a, b, precision=lax.Precision.HIGHEST,
                   preferred_element_type=jnp.float32)


def _sage_body(p_ref, c_ref, x_ref, wl_ref, wr_ref, b_ref, o_ref):
    ssum = p_ref[0] + p_ref[1]
    cnt = c_ref[0, :, 0] + c_ref[1, :, 0]
    mean = ssum / jnp.maximum(cnt, 1.0)[:, None]
    o_ref[...] = _dot(mean, wl_ref[...]) + _dot(x_ref[...], wr_ref[...]) + b_ref[0:1, :]


def _sage_res_body(p_ref, c_ref, x_ref, wl_ref, wr_ref, b_ref, r_ref, o_ref):
    ssum = p_ref[0] + p_ref[1]
    cnt = c_ref[0, :, 0] + c_ref[1, :, 0]
    mean = ssum / jnp.maximum(cnt, 1.0)[:, None]
    o_ref[...] = (_dot(mean, wl_ref[...]) + _dot(x_ref[...], wr_ref[...])
                  + b_ref[0:1, :] + r_ref[...])


def _tc_sage(T, residual=False):
    grid = (T // _R,)
    in_specs = [
        pl.BlockSpec((2, _R, D), lambda i: (0, i, 0)),
        pl.BlockSpec((2, _R, 16), lambda i: (0, i, 0)),
        pl.BlockSpec((_R, D), lambda i: (i, 0)),
        pl.BlockSpec((D, D), lambda i: (0, 0)),
        pl.BlockSpec((D, D), lambda i: (0, 0)),
        pl.BlockSpec((8, D), lambda i: (0, 0)),
    ]
    body = _sage_body
    if residual:
        in_specs.append(pl.BlockSpec((_R, D), lambda i: (i, 0)))
        body = _sage_res_body
    return pl.pallas_call(
        body,
        grid=grid,
        in_specs=in_specs,
        out_specs=pl.BlockSpec((_R, D), lambda i: (i, 0)),
        out_shape=jax.ShapeDtypeStruct((T, D), jnp.float32),
    )


_sage_N = _tc_sage(T8)
_sage_N_res = _tc_sage(T8, residual=True)


def _wconv_score_body(p_ref, b_ref, pn_ref, hw_ref, sc_ref):
    hw = b_ref[:, 0:1] * (p_ref[0] + p_ref[1])
    hw_ref[...] = hw
    sc_ref[...] = jnp.tanh(_dot(hw, pn_ref[...]))


_wconv_score = pl.pallas_call(
    _wconv_score_body,
    grid=(T8 // _R,),
    in_specs=[
        pl.BlockSpec((2, _R, D), lambda i: (0, i, 0)),
        pl.BlockSpec((_R, 16), lambda i: (i, 0)),
        pl.BlockSpec((D, 16), lambda i: (0, 0)),
    ],
    out_specs=[
        pl.BlockSpec((_R, D), lambda i: (i, 0)),
        pl.BlockSpec((_R, 16), lambda i: (i, 0)),
    ],
    out_shape=[
        jax.ShapeDtypeStruct((T8, D), jnp.float32),
        jax.ShapeDtypeStruct((T8, 16), jnp.float32),
    ],
)


def _acomb_body(p_ref, a_ref, o_ref):
    o_ref[...] = a_ref[:, 0:1] * (p_ref[0] + p_ref[1])


_acomb = pl.pallas_call(
    _acomb_body,
    grid=(T8 // _R,),
    in_specs=[
        pl.BlockSpec((2, _R, D), lambda i: (0, i, 0)),
        pl.BlockSpec((_R, 16), lambda i: (i, 0)),
    ],
    out_specs=pl.BlockSpec((_R, D), lambda i: (i, 0)),
    out_shape=jax.ShapeDtypeStruct((T8, D), jnp.float32),
)


def _col16(v, T):
    out = jnp.zeros((T, 16), jnp.float32)
    return out.at[:, 0].set(v)


def kernel(x, edge_index, batch, weights,
           Wl_d00, bl_d00, Wr_d00, Wl_d01, bl_d01, Wr_d01,
           Wl_bt0, bl_bt0, Wr_bt0, Wl_bt1, bl_bt1, Wr_bt1,
           Wl_u00, bl_u00, Wr_u00, Wl_u01, bl_u01, Wr_u01, pool_p):
    f32 = jnp.float32
    src = edge_index[0].astype(jnp.int32)
    dst = edge_index[1].astype(jnp.int32)
    pad = N + (jnp.arange(EP - E, dtype=jnp.int32) % (T8 - N))
    srcp = jnp.concatenate([src, pad])
    dstp = jnp.concatenate([dst, pad])

    def g3(idx):
        r = idx.reshape(NW, NSTAGE, IB, B)
        return jnp.concatenate(
            [r, jnp.zeros((NW, NSTAGE, 2, B), jnp.int32)], axis=2)

    def s3(idx):
        return idx.reshape(NW, NSTAGE, IB, B)

    src_g, src_s = g3(srcp), s3(srcp)
    dst_g, dst_s = g3(dstp), s3(dstp)

    x_pad = jnp.zeros((T8, D), f32).at[:N].set(x)
    _Z128_N = jnp.zeros((T8, D), f32)
    _Z16_N = jnp.zeros((T8, 16), f32)

    def bias8(b):
        return jnp.broadcast_to(b[None, :], (8, D))

    Cdst = _make_count_hist(T8)(dst_s, _Z16_N)
    Cdeg = _make_count_hist(T8)(src_s, _Z16_N)

    P = _edge128_N(x_pad, src_g, dst_s, _Z128_N)
    h1 = _sage_N(P, Cdst, x_pad, Wl_d00.T, Wr_d00.T, bias8(bl_d00))
    P = _edge128_N(h1, src_g, dst_s, _Z128_N)
    h2 = _sage_N(P, Cdst, h1, Wl_d01.T, Wr_d01.T, bias8(bl_d01))

    deg = Cdeg[0, :, 0] + Cdeg[1, :, 0]
    w0p = jnp.zeros((T8,), f32).at[:N].set(weights[:, 0])
    a = w0p / deg
    a_tab = jnp.zeros((T8, D), f32).at[:, 0].set(a)
    Ca = _edge128_N(a_tab, src_g, dst_s, _Z128_N)
    aggr = Ca[0, :, 0] + Ca[1, :, 0] + 1e-12
    b = 1.0 / aggr

    pn = pool_p / jnp.linalg.norm(pool_p)
    pn16 = jnp.zeros((D, 16), f32).at[:, 0].set(pn)
    Pw = _edge128_N(a[:, None] * h2, src_g, dst_s, _Z128_N)
    hw, score16 = _wconv_score(Pw, _col16(b, T8), pn16)
    score = score16[:N, 0]

    topv, _perm = lax.top_k(score, K)
    tau = topv[K - 1]
    gt = score > tau
    n_gt = jnp.sum(gt.astype(jnp.int32))
    tie = score == tau
    tie_rank = jnp.cumsum(tie.astype(jnp.int32))
    keep = gt | (tie & (tie_rank <= (K - n_gt)))
    keepf = jnp.zeros((T8,), f32).at[:N].set(keep.astype(f32))

    hp_pad = hw * (score16[:, 0] * keepf)[:, None]

    keep_tab = jnp.zeros((T8, D), f32).at[:, 0].set(keepf)
    C2p = _edge128_N(keep_tab, src_g, dst_s, _Z128_N)
    C2 = C2p[:, :, :16]

    P = _edge128_N(hp_pad, src_g, dst_s, _Z128_N)
    hb = _sage_N(P, C2, hp_pad, Wl_bt0.T, Wr_bt0.T, bias8(bl_bt0))
    hb = hb * keepf[:, None]
    P = _edge128_N(hb, src_g, dst_s, _Z128_N)
    hb = _sage_N(P, C2, hb, Wl_bt1.T, Wr_bt1.T, bias8(bl_bt1))

    hu_nodes = hb * keepf[:, None]
    Pr = _edge128_N(b[:, None] * hu_nodes, dst_g, src_s, _Z128_N)
    hu = _acomb(Pr, _col16(a, T8))

    P = _edge128_N(hu, src_g, dst_s, _Z128_N)
    h = _sage_N(P, Cdst, hu, Wl_u00.T, Wr_u00.T, bias8(bl_u00))
    P = _edge128_N(h, src_g, dst_s, _Z128_N)
    out = _sage_N_res(P, Cdst, h, Wl_u01.T, Wr_u01.T, bias8(bl_u01), h2)
    return out[:N]
